# baseline (speedup 1.0000x reference)
import functools

import jax
import jax.numpy as jnp
from jax import lax
from jax.experimental import pallas as pl
from jax.experimental.pallas import tpu as pltpu
from jax.experimental.pallas import tpu_sc as plsc

_LANES = 16


def _sc_info():
    try:
        info = plsc.get_sparse_core_info()
        return info.num_cores, info.num_subcores
    except Exception:
        return 2, 16


@functools.lru_cache(maxsize=None)
def _make_sc_kernel(B, L, D, KP, NM, NC, NS):
    NW = NC * NS
    RPW = B // NW
    G = 4
    NB = RPW // G
    GL = G * L
    GKP = G * KP
    NKC = KP // _LANES
    MIX_N = RPW * KP
    GLP = 896

    mesh = plsc.VectorSubcoreMesh(core_axis_name="c", subcore_axis_name="s")

    def body(ida_hbm, idb_hbm, mix_hbm, rat_hbm, table_hbm, out_hbm,
             mixL, ratL, mixS, ratS, klastB, mT,
             idaC0, idaC1, idbC0, idbC1, selB0, selB1,
             rowsA0, rowsA1, rowsB0, rowsB1,
             semA0, semA1, semB0, semB1, semO0, semO1, semI0, semI1):
        wid = lax.axis_index("s") * NC + lax.axis_index("c")
        base = wid * (RPW * L)

        iota = lax.iota(jnp.int32, _LANES)

        pltpu.sync_copy(mix_hbm.at[pl.ds(wid * (RPW * NM), RPW * NM)], mixS)
        pltpu.sync_copy(rat_hbm.at[pl.ds(wid * (RPW * NM), RPW * NM)], ratS)

        def repack(r, carry):
            for kc in range(NKC):
                lane = kc * _LANES + iota
                if (kc + 1) * _LANES <= NM:
                    idx = r * NM + lane
                    mixL[pl.ds(r * KP + kc * _LANES, _LANES)] = (
                        plsc.load_gather(mixS, [idx]))
                    ratL[pl.ds(r * KP + kc * _LANES, _LANES)] = (
                        plsc.load_gather(ratS, [idx]))
                else:
                    valid = lane < NM
                    idx = jnp.where(valid, r * NM + lane, r * NM)
                    mv = plsc.load_gather(mixS, [idx])
                    rv = plsc.load_gather(ratS, [idx])
                    mixL[pl.ds(r * KP + kc * _LANES, _LANES)] = (
                        jnp.where(valid, mv, L))
                    ratL[pl.ds(r * KP + kc * _LANES, _LANES)] = (
                        jnp.where(valid, rv, 1.0))
            return carry
        lax.fori_loop(0, RPW, repack, 0)

        z16 = jnp.zeros((_LANES,), jnp.int32)
        for q in range((GLP - GL) // _LANES):
            idaC0[pl.ds(GL + q * _LANES, _LANES)] = z16
            idaC1[pl.ds(GL + q * _LANES, _LANES)] = z16
        idbC0[pl.ds(GL, _LANES)] = z16
        idbC1[pl.ds(GL, _LANES)] = z16

        def stage_start(jn, idaC, idbC, semI):
            src = base + jn * GL
            pltpu.make_async_copy(
                ida_hbm.at[pl.ds(src, GL)], idaC.at[pl.ds(0, GL)], semI).start()
            pltpu.make_async_copy(
                idb_hbm.at[pl.ds(src, GL)], idbC.at[pl.ds(0, GL)], semI).start()

        def stage_wait(jn, idaC, idbC, semI):
            src = base + jn * GL
            pltpu.make_async_copy(
                ida_hbm.at[pl.ds(src, GL)], idaC.at[pl.ds(0, GL)], semI).wait()
            pltpu.make_async_copy(
                idb_hbm.at[pl.ds(src, GL)], idbC.at[pl.ds(0, GL)], semI).wait()

        A_CHUNKS = [(c * 128, 128) for c in range(GLP // 128)]
        B_CHUNKS = [(h * 128, 128) for h in range(GKP // 128)]

        def issue(jn, idaC, idbC, selB, rowsA, rowsB, semA, semB):
            for (co, cl) in A_CHUNKS:
                pltpu.make_async_copy(
                    table_hbm.at[idaC.at[pl.ds(co, cl)]],
                    rowsA.at[pl.ds(co, cl)], semA).start()
            mloc = jn * GKP
            for t in range(G):
                for kc in range(NKC):
                    mv = mixL[pl.ds(mloc + t * KP + kc * _LANES, _LANES)]
                    selB[pl.ds(t * KP + kc * _LANES, _LANES)] = (
                        plsc.load_gather(idbC, [mv + t * L]))
            for (co, cl) in B_CHUNKS:
                pltpu.make_async_copy(
                    table_hbm.at[selB.at[pl.ds(co, cl)]],
                    rowsB.at[pl.ds(co, cl)], semB).start()

        def wait_gathers(idaC, selB, rowsA, rowsB, semA, semB):
            for (co, cl) in A_CHUNKS:
                pltpu.make_async_copy(
                    table_hbm.at[idaC.at[pl.ds(co, cl)]],
                    rowsA.at[pl.ds(co, cl)], semA).wait()
            for (co, cl) in B_CHUNKS:
                pltpu.make_async_copy(
                    table_hbm.at[selB.at[pl.ds(co, cl)]],
                    rowsB.at[pl.ds(co, cl)], semB).wait()

        def klast_batch(jn):
            def row(t, carry):
                mloc = jn * GKP + t * KP
                mixvs = [mixL[pl.ds(mloc + kc * _LANES, _LANES)]
                         for kc in range(NKC)]
                kls = [iota + kc * _LANES for kc in range(NKC)]
                for k in range(NM):
                    sp = jnp.full((_LANES,), mloc + k, jnp.int32)
                    mk = plsc.load_gather(mixL, [sp])
                    for kc in range(NKC):
                        kls[kc] = jnp.where(mixvs[kc] == mk, k, kls[kc])
                for kc in range(NKC):
                    klastB[pl.ds(t * KP + kc * _LANES, _LANES)] = kls[kc]
                return carry
            lax.fori_loop(0, G, row, 0)

        def blend_batch(jn, rowsA, rowsB):
            def row(t, carry):
                mloc = jn * GKP + t * KP
                metas = []
                for kc in range(NKC):
                    mv = mixL[pl.ds(mloc + kc * _LANES, _LANES)]
                    kl = klastB[pl.ds(t * KP + kc * _LANES, _LANES)]
                    kvec = iota + kc * _LANES
                    keep = (kl == kvec) & (mv < L)
                    rv = ratL[pl.ds(mloc + kc * _LANES, _LANES)]
                    metas.append((mv + t * L, kvec + t * KP, keep, rv, 1.0 - rv))
                for kc in range(NKC):
                    mv2, kvec2, keep, rv, omrv = metas[kc]
                    for d in range(D):
                        dsp = jnp.full((_LANES,), d, jnp.int32)
                        a = plsc.load_gather(rowsA, [mv2, dsp])
                        bv = plsc.load_gather(rowsB, [kvec2, dsp])
                        mT[d, pl.ds(kc * _LANES, _LANES)] = rv * a + omrv * bv
                for kc in range(NKC):
                    mv2, kvec2, keep, rv, omrv = metas[kc]
                    for d in range(D):
                        dsp = jnp.full((_LANES,), d, jnp.int32)
                        m = mT[d, pl.ds(kc * _LANES, _LANES)]
                        plsc.store_scatter(rowsA, [mv2, dsp], m, mask=keep)
                return carry
            lax.fori_loop(0, G, row, 0)

        def start_out(jn, rowsA, semO):
            pltpu.make_async_copy(
                rowsA.at[pl.ds(0, GL)],
                out_hbm.at[pl.ds(base + jn * GL, GL)], semO).start()

        def wait_out(jn, rowsA, semO):
            pltpu.make_async_copy(
                rowsA.at[pl.ds(0, GL)],
                out_hbm.at[pl.ds(base + jn * GL, GL)], semO).wait()

        bufs0 = (idaC0, selB0, rowsA0, rowsB0, semA0, semB0)
        bufs1 = (idaC1, selB1, rowsA1, rowsB1, semA1, semB1)
        SMAX = NB // 2 - 1

        stage_start(0, idaC0, idbC0, semI0)
        stage_wait(0, idaC0, idbC0, semI0)
        issue(0, idaC0, idbC0, selB0, rowsA0, rowsB0, semA0, semB0)
        stage_start(1, idaC1, idbC1, semI1)

        def super_body(s, carry):
            j0 = 2 * s
            j1 = j0 + 1
            klast_batch(j0)
            @pl.when(s > 0)
            def _():
                wait_out(j0 - 1, rowsA1, semO1)
            stage_wait(j1, idaC1, idbC1, semI1)
            issue(j1, idaC1, idbC1, selB1, rowsA1, rowsB1, semA1, semB1)
            wait_gathers(*bufs0)
            @pl.when(s < SMAX)
            def _():
                stage_start(j0 + 2, idaC0, idbC0, semI0)
            blend_batch(j0, rowsA0, rowsB0)
            start_out(j0, rowsA0, semO0)
            klast_batch(j1)
            wait_gathers(*bufs1)
            @pl.when(s < SMAX)
            def _():
                stage_start(j1 + 2, idaC1, idbC1, semI1)
            blend_batch(j1, rowsA1, rowsB1)
            start_out(j1, rowsA1, semO1)
            @pl.when(s < SMAX)
            def _():
                wait_out(j0, rowsA0, semO0)
                stage_wait(j0 + 2, idaC0, idbC0, semI0)
                issue(j0 + 2, idaC0, idbC0, selB0, rowsA0, rowsB0,
                      semA0, semB0)
            return carry

        lax.fori_loop(0, NB // 2, super_body, 0)
        wait_out(NB - 2, rowsA0, semO0)
        wait_out(NB - 1, rowsA1, semO1)

    return pl.kernel(
        body,
        out_type=jax.ShapeDtypeStruct((B * L, D), jnp.float32),
        mesh=mesh,
        compiler_params=pltpu.CompilerParams(needs_layout_passes=False,
                                             use_tc_tiling_on_sc=False),
        scratch_types=[
            pltpu.VMEM((MIX_N,), jnp.int32),
            pltpu.VMEM((MIX_N,), jnp.float32),
            pltpu.VMEM((RPW * NM,), jnp.int32),
            pltpu.VMEM((RPW * NM,), jnp.float32),
            pltpu.VMEM((GKP,), jnp.int32),
            pltpu.VMEM((D, KP), jnp.float32),
            pltpu.VMEM((GLP,), jnp.int32),
            pltpu.VMEM((GLP,), jnp.int32),
            pltpu.VMEM((GL + _LANES,), jnp.int32),
            pltpu.VMEM((GL + _LANES,), jnp.int32),
            pltpu.VMEM((GKP,), jnp.int32),
            pltpu.VMEM((GKP,), jnp.int32),
            pltpu.VMEM((GLP, D), jnp.float32),
            pltpu.VMEM((GLP, D), jnp.float32),
            pltpu.VMEM((GKP, D), jnp.float32),
            pltpu.VMEM((GKP, D), jnp.float32),
            pltpu.SemaphoreType.DMA,
            pltpu.SemaphoreType.DMA,
            pltpu.SemaphoreType.DMA,
            pltpu.SemaphoreType.DMA,
            pltpu.SemaphoreType.DMA,
            pltpu.SemaphoreType.DMA,
            pltpu.SemaphoreType.DMA,
            pltpu.SemaphoreType.DMA,
        ],
    )


def kernel(input_ids_a, input_ids_b, mix_idxes, mix_ratios, table):
    B, L = input_ids_a.shape
    NM = mix_idxes.shape[1]
    D = table.shape[1]
    KP = 64
    NC, NS = _sc_info()

    kern = _make_sc_kernel(B, L, D, KP, NM, NC, NS)

    ida = input_ids_a.astype(jnp.int32).reshape(-1)
    idb = input_ids_b.astype(jnp.int32).reshape(-1)
    mixf = mix_idxes.astype(jnp.int32).reshape(-1)
    ratf = mix_ratios.astype(jnp.float32).reshape(-1)

    return kern(ida, idb, mixf, ratf, table).reshape(B, L, D)


# speedup vs baseline: 1.5611x; 1.5611x over previous
import functools

import jax
import jax.numpy as jnp
from jax import lax
from jax.experimental import pallas as pl
from jax.experimental.pallas import tpu as pltpu
from jax.experimental.pallas import tpu_sc as plsc

_LANES = 16


def _sc_info():
    try:
        info = plsc.get_sparse_core_info()
        return info.num_cores, info.num_subcores
    except Exception:
        return 2, 16


@functools.lru_cache(maxsize=None)
def _make_sc_kernel(B, L, D, KP, NM, NC, NS):
    NW = NC * NS
    RPW = B // NW
    G = 4
    NB = RPW // G
    GL = G * L
    GKP = G * KP
    NKC = KP // _LANES
    MIX_N = RPW * KP
    A_CHUNKS = []
    off = 0
    while off < GL:
        ln = min(128, GL - off)
        A_CHUNKS.append((off, ln))
        off += ln
    B_CHUNKS = [(h * 128, 128) for h in range(GKP // 128)]

    mesh = plsc.VectorSubcoreMesh(core_axis_name="c", subcore_axis_name="s")

    def body(ida_hbm, idb_hbm, mix_hbm, rat_hbm, table_hbm, out_hbm,
             mixL, ratL, klastB,
             idaC0, idaC1, idbC0, idbC1, selB0, selB1,
             rowsA0, rowsA1, rowsB0, rowsB1,
             semA0, semA1, semB0, semB1, semO0, semO1):
        wid = lax.axis_index("s") * NC + lax.axis_index("c")
        base = wid * (RPW * L)
        mbase = wid * MIX_N

        pltpu.sync_copy(mix_hbm.at[pl.ds(mbase, MIX_N)], mixL)
        pltpu.sync_copy(rat_hbm.at[pl.ds(mbase, MIX_N)], ratL)
        z16 = jnp.zeros((_LANES,), jnp.int32)
        idbC0[pl.ds(GL, _LANES)] = z16
        idbC1[pl.ds(GL, _LANES)] = z16

        iota = lax.iota(jnp.int32, _LANES)

        def stage_issue(jn, idaC, idbC, selB, rowsA, rowsB, semA, semB):
            src = base + jn * GL
            pltpu.sync_copy(ida_hbm.at[pl.ds(src, GL)], idaC)
            pltpu.sync_copy(idb_hbm.at[pl.ds(src, GL)], idbC.at[pl.ds(0, GL)])
            for (co, cl) in A_CHUNKS:
                pltpu.make_async_copy(
                    table_hbm.at[idaC.at[pl.ds(co, cl)]],
                    rowsA.at[pl.ds(co, cl)], semA).start()
            mloc = jn * GKP
            for t in range(G):
                for kc in range(NKC):
                    mv = mixL[pl.ds(mloc + t * KP + kc * _LANES, _LANES)]
                    selB[pl.ds(t * KP + kc * _LANES, _LANES)] = (
                        plsc.load_gather(idbC, [mv + t * L]))
            for (co, cl) in B_CHUNKS:
                pltpu.make_async_copy(
                    table_hbm.at[selB.at[pl.ds(co, cl)]],
                    rowsB.at[pl.ds(co, cl)], semB).start()

        def wait_gathers(idaC, selB, rowsA, rowsB, semA, semB):
            for (co, cl) in A_CHUNKS:
                pltpu.make_async_copy(
                    table_hbm.at[idaC.at[pl.ds(co, cl)]],
                    rowsA.at[pl.ds(co, cl)], semA).wait()
            for (co, cl) in B_CHUNKS:
                pltpu.make_async_copy(
                    table_hbm.at[selB.at[pl.ds(co, cl)]],
                    rowsB.at[pl.ds(co, cl)], semB).wait()

        def klast_batch(jn):
            def row(t, carry):
                mloc = jn * GKP + t * KP
                mixvs = [mixL[pl.ds(mloc + kc * _LANES, _LANES)]
                         for kc in range(NKC)]
                kls = [iota + kc * _LANES for kc in range(NKC)]
                for k in range(NM):
                    sp = jnp.full((_LANES,), mloc + k, jnp.int32)
                    mk = plsc.load_gather(mixL, [sp])
                    for kc in range(NKC):
                        kls[kc] = jnp.where(mixvs[kc] == mk, k, kls[kc])
                for kc in range(NKC):
                    klastB[pl.ds(t * KP + kc * _LANES, _LANES)] = kls[kc]
                return carry
            lax.fori_loop(0, G, row, 0)

        def blend_batch(jn, rowsA, rowsB):
            def row(t, carry):
                mloc = jn * GKP + t * KP
                for kc in range(NKC):
                    mv = mixL[pl.ds(mloc + kc * _LANES, _LANES)]
                    kl = klastB[pl.ds(t * KP + kc * _LANES, _LANES)]
                    kvec = iota + kc * _LANES
                    keep = (kl == kvec) & (mv < L)
                    rv = ratL[pl.ds(mloc + kc * _LANES, _LANES)]
                    omrv = 1.0 - rv
                    mv2 = mv + t * L
                    kvec2 = kvec + t * KP
                    for d in range(D):
                        dsp = jnp.full((_LANES,), d, jnp.int32)
                        a = plsc.load_gather(rowsA, [mv2, dsp])
                        bv = plsc.load_gather(rowsB, [kvec2, dsp])
                        plsc.store_scatter(rowsA, [mv2, dsp],
                                           rv * a + omrv * bv, mask=keep)
                return carry
            lax.fori_loop(0, G, row, 0)

        def start_out(jn, rowsA, semO):
            pltpu.make_async_copy(
                rowsA.at[pl.ds(0, GL)],
                out_hbm.at[pl.ds(base + jn * GL, GL)], semO).start()

        def wait_out(jn, rowsA, semO):
            pltpu.make_async_copy(
                rowsA.at[pl.ds(0, GL)],
                out_hbm.at[pl.ds(base + jn * GL, GL)], semO).wait()

        bufs0 = (idaC0, selB0, rowsA0, rowsB0, semA0, semB0)
        bufs1 = (idaC1, selB1, rowsA1, rowsB1, semA1, semB1)

        stage_issue(0, idaC0, idbC0, selB0, rowsA0, rowsB0, semA0, semB0)

        def super_body(s, carry):
            j0 = 2 * s
            klast_batch(j0)
            @pl.when(s > 0)
            def _():
                wait_out(j0 - 1, rowsA1, semO1)
            stage_issue(j0 + 1, idaC1, idbC1, selB1, rowsA1, rowsB1,
                        semA1, semB1)
            wait_gathers(*bufs0)
            blend_batch(j0, rowsA0, rowsB0)
            start_out(j0, rowsA0, semO0)
            klast_batch(j0 + 1)
            wait_gathers(*bufs1)
            blend_batch(j0 + 1, rowsA1, rowsB1)
            start_out(j0 + 1, rowsA1, semO1)
            @pl.when(s < NB // 2 - 1)
            def _():
                wait_out(j0, rowsA0, semO0)
                stage_issue(j0 + 2, idaC0, idbC0, selB0, rowsA0, rowsB0,
                            semA0, semB0)
            return carry

        lax.fori_loop(0, NB // 2, super_body, 0)
        wait_out(NB - 2, rowsA0, semO0)
        wait_out(NB - 1, rowsA1, semO1)

    return pl.kernel(
        body,
        out_type=jax.ShapeDtypeStruct((B * L, D), jnp.float32),
        mesh=mesh,
        compiler_params=pltpu.CompilerParams(needs_layout_passes=False,
                                             use_tc_tiling_on_sc=False),
        scratch_types=[
            pltpu.VMEM((MIX_N,), jnp.int32),
            pltpu.VMEM((MIX_N,), jnp.float32),
            pltpu.VMEM((GKP,), jnp.int32),
            pltpu.VMEM((GL,), jnp.int32),
            pltpu.VMEM((GL,), jnp.int32),
            pltpu.VMEM((GL + _LANES,), jnp.int32),
            pltpu.VMEM((GL + _LANES,), jnp.int32),
            pltpu.VMEM((GKP,), jnp.int32),
            pltpu.VMEM((GKP,), jnp.int32),
            pltpu.VMEM((GL + _LANES, D), jnp.float32),
            pltpu.VMEM((GL + _LANES, D), jnp.float32),
            pltpu.VMEM((GKP, D), jnp.float32),
            pltpu.VMEM((GKP, D), jnp.float32),
            pltpu.SemaphoreType.DMA,
            pltpu.SemaphoreType.DMA,
            pltpu.SemaphoreType.DMA,
            pltpu.SemaphoreType.DMA,
            pltpu.SemaphoreType.DMA,
            pltpu.SemaphoreType.DMA,
        ],
    )


def kernel(input_ids_a, input_ids_b, mix_idxes, mix_ratios, table):
    B, L = input_ids_a.shape
    NM = mix_idxes.shape[1]
    D = table.shape[1]
    KP = 64
    NC, NS = _sc_info()

    kern = _make_sc_kernel(B, L, D, KP, NM, NC, NS)

    ida = input_ids_a.astype(jnp.int32).reshape(-1)
    idb = input_ids_b.astype(jnp.int32).reshape(-1)
    mixp = jnp.concatenate(
        [mix_idxes.astype(jnp.int32),
         jnp.full((B, KP - NM), L, jnp.int32)], axis=1).reshape(-1)
    ratp = jnp.concatenate(
        [mix_ratios.astype(jnp.float32),
         jnp.ones((B, KP - NM), jnp.float32)], axis=1).reshape(-1)

    return kern(ida, idb, mixp, ratp, table).reshape(B, L, D)
